# Initial kernel scaffold; baseline (speedup 1.0000x reference)
import jax
import jax.numpy as jnp
from jax.experimental import pallas as pl


def kernel(x, edge_index, batch, params):
    raise NotImplementedError("write your pallas kernel here")


# speedup vs baseline: 1.6164x; 1.6164x over previous
import functools

import jax
import jax.numpy as jnp
from jax import lax
from jax.experimental import pallas as pl
from jax.experimental.pallas import tpu as pltpu
from jax.experimental.pallas import tpu_sc as plsc

N_NODES = 10000
N_EDGES = 320000
HID = 128
N_ITERS = 4

NC = 2
NS = 16
NW = NC * NS
CHUNK = 128
NCHUNKS = N_EDGES // CHUNK
CHUNKS_PER_W = -(-NCHUNKS // NW)

EBLK = 2000
NBLK = 2000

_f32 = jnp.float32


def _tc_call(body, blocked, consts, out_minors, rows, blk):
    grid = (rows // blk,)
    in_specs = (
        [pl.BlockSpec((blk, a.shape[-1]), lambda i: (i, 0)) for a in blocked]
        + [pl.BlockSpec(c.shape, lambda i: (0, 0)) for c in consts]
    )
    out_specs = [pl.BlockSpec((blk, m), lambda i: (i, 0)) for m in out_minors]
    out_shape = [jax.ShapeDtypeStruct((rows, m), _f32) for m in out_minors]
    return pl.pallas_call(
        body,
        grid=grid,
        in_specs=in_specs,
        out_specs=out_specs,
        out_shape=out_shape,
        compiler_params=pltpu.CompilerParams(
            dimension_semantics=("arbitrary",),
        ),
    )(*blocked, *consts)


def _dot(a, b):
    return jnp.dot(a, b, preferred_element_type=_f32)


def _encode_body(x, w1, b1, w2, b2, h_o):
    z = jnp.maximum(_dot(x[...], w1[...]) + b1[...], 0.0)
    h_o[...] = _dot(z, w2[...]) + b2[...]


def _edge_encode_body(hs, he, w1, b1, w2, b2, e_o):
    z = jnp.concatenate([hs[...], he[...]], axis=-1)
    z1 = jnp.maximum(_dot(z, w1[...]) + b1[...], 0.0)
    e_o[...] = _dot(z1, w2[...]) + b2[...]


def _edge_update_body(hs, he, e, w1, b1, w2, b2, ef_o, eo_o):
    z = jnp.concatenate([hs[...], he[...], e[...]], axis=-1)
    z1 = jnp.maximum(_dot(z, w1[...]) + b1[...], 0.0)
    ef = _dot(z1, w2[...]) + b2[...]
    ef_o[...] = ef
    eo_o[...] = e[...] + ef


def _node_update_body(h, m, w1, b1, w2, b2, h_o):
    z = jnp.concatenate([h[...], m[...]], axis=-1)
    z1 = jnp.maximum(_dot(z, w1[...]) + b1[...], 0.0)
    h_o[...] = h[...] + _dot(z1, w2[...]) + b2[...]


def _scores_body(hs, he, e, w1, b1, w2, b2, w3, b3, s_o):
    z = jnp.concatenate([hs[...], he[...], e[...]], axis=-1)
    z1 = jnp.maximum(_dot(z, w1[...]) + b1[...], 0.0)
    z2 = jnp.maximum(_dot(z1, w2[...]) + b2[...], 0.0)
    s_o[...] = jax.nn.sigmoid(_dot(z2, w3[...]) + b3[...])


def _features_body(h, w1, b1, w2, b2, w3, b3, f_o):
    z1 = jnp.maximum(_dot(h[...], w1[...]) + b1[...], 0.0)
    z2 = jnp.maximum(_dot(z1, w2[...]) + b2[...], 0.0)
    f_o[...] = _dot(z2, w3[...]) + b3[...]


@functools.lru_cache(maxsize=None)
def _make_sc_gather(npairs):
    mesh = plsc.VectorSubcoreMesh(core_axis_name="c", subcore_axis_name="s")
    out_type = [jax.ShapeDtypeStruct((N_EDGES, HID), _f32) for _ in range(npairs)]
    scratch = [
        pltpu.VMEM((CHUNK,), jnp.int32),
        pltpu.VMEM((CHUNK, HID), _f32),
        pltpu.SemaphoreType.DMA,
    ]

    @functools.partial(pl.kernel, mesh=mesh, out_type=out_type,
                       scratch_types=scratch)
    def gather_kernel(*refs):
        tables = refs[:npairs]
        idxs = refs[npairs:2 * npairs]
        outs = refs[2 * npairs:3 * npairs]
        idx_v, rows_v, sem = refs[3 * npairs:]
        wid = lax.axis_index("s") * NC + lax.axis_index("c")

        def body(j, carry):
            base = (wid + j * NW) * CHUNK

            @pl.when(base < N_EDGES)
            def _():
                for p in range(npairs):
                    pltpu.sync_copy(idxs[p].at[pl.ds(base, CHUNK)], idx_v)
                    pltpu.async_copy(tables[p].at[idx_v], rows_v, sem).wait()
                    pltpu.sync_copy(rows_v, outs[p].at[pl.ds(base, CHUNK)])

            return carry

        lax.fori_loop(0, CHUNKS_PER_W, body, 0)

    return gather_kernel


def kernel(x, edge_index, batch, params):
    start, end = edge_index[0], edge_index[1]
    r = lambda b: b.reshape(1, -1)

    (wn1, bn1), (wn2, bn2) = params["node_encoder"]
    (we1, be1), (we2, be2) = params["edge_encoder"]
    (wu1, bu1), (wu2, bu2) = params["edge_net"]
    (wf1, bf1), (wf2, bf2) = params["feature_net"]
    fo = params["features_out"]
    so = params["scores_out"]

    (h,) = _tc_call(_encode_body, [x], [wn1, r(bn1), wn2, r(bn2)],
                    [HID], N_NODES, NBLK)
    hs, he = _make_sc_gather(2)(h, h, start, end)

    (e,) = _tc_call(_edge_encode_body, [hs, he],
                    [we1, r(be1), we2, r(be2)], [HID], N_EDGES, EBLK)

    for it in range(N_ITERS):
        ef, e = _tc_call(_edge_update_body, [hs, he, e],
                         [wu1, r(bu1), wu2, r(bu2)], [HID, HID],
                         N_EDGES, EBLK)
        m = jax.ops.segment_sum(ef, end, num_segments=N_NODES)
        (h,) = _tc_call(_node_update_body, [h, m],
                        [wf1, r(bf1), wf2, r(bf2)], [HID], N_NODES, NBLK)
        hs, he = _make_sc_gather(2)(h, h, start, end)

    (scores,) = _tc_call(
        _scores_body, [hs, he, e],
        [so[0][0], r(so[0][1]), so[1][0], r(so[1][1]), so[2][0], r(so[2][1])],
        [1], N_EDGES, EBLK)

    (feats,) = _tc_call(
        _features_body, [h],
        [fo[0][0], r(fo[0][1]), fo[1][0], r(fo[1][1]), fo[2][0], r(fo[2][1])],
        [HID], N_NODES, NBLK)

    return feats, scores
